# baseline (speedup 1.0000x reference)
import functools

import jax
import jax.numpy as jnp
from jax import lax
from jax.experimental import pallas as pl
from jax.experimental.pallas import tpu as pltpu
from jax.experimental.pallas import tpu_sc as plsc

NUM_ENTITIES = 1000000
NUM_RELATIONS = 1000
EMB_DIM = 32
BATCH = 16384

NC = 2
NS = 16
NW = NC * NS
BPW = BATCH // NW
CH = 128
NCH = BPW // CH


def _tec_body(head_hbm, rel_hbm, tail_hbm, ent_hbm, relt_hbm,
              oh_hbm, or_hbm, ot_hbm,
              idx_h, idx_r, idx_t, rows_h, rows_r, rows_t,
              sem_h, sem_r, sem_t):
  wid = lax.axis_index("s") * NC + lax.axis_index("c")

  pltpu.sync_copy(head_hbm.at[wid], idx_h)
  pltpu.sync_copy(rel_hbm.at[wid], idx_r)
  pltpu.sync_copy(tail_hbm.at[wid], idx_t)

  waits = []
  for j in range(NCH):
    waits.append(pltpu.async_copy(ent_hbm.at[idx_h.at[j]], rows_h.at[j], sem_h))
    waits.append(pltpu.async_copy(relt_hbm.at[idx_r.at[j]], rows_r.at[j], sem_r))
    waits.append(pltpu.async_copy(ent_hbm.at[idx_t.at[j]], rows_t.at[j], sem_t))
  for w in waits:
    w.wait()

  pltpu.sync_copy(rows_h, oh_hbm.at[wid])
  pltpu.sync_copy(rows_r, or_hbm.at[wid])
  pltpu.sync_copy(rows_t, ot_hbm.at[wid])


@jax.jit
def _transe_lookup(head, rel, tail, entity_table, relation_table):
  mesh = plsc.VectorSubcoreMesh(core_axis_name="c", subcore_axis_name="s")
  out_t = jax.ShapeDtypeStruct((NW, NCH, CH, EMB_DIM), jnp.float32)
  run = pl.kernel(
      _tec_body,
      out_type=(out_t, out_t, out_t),
      mesh=mesh,
      scratch_types=[
          pltpu.VMEM((NCH, CH), jnp.int32),
          pltpu.VMEM((NCH, CH), jnp.int32),
          pltpu.VMEM((NCH, CH), jnp.int32),
          pltpu.VMEM((NCH, CH, EMB_DIM), jnp.float32),
          pltpu.VMEM((NCH, CH, EMB_DIM), jnp.float32),
          pltpu.VMEM((NCH, CH, EMB_DIM), jnp.float32),
          pltpu.SemaphoreType.DMA,
          pltpu.SemaphoreType.DMA,
          pltpu.SemaphoreType.DMA,
      ],
      compiler_params=pltpu.CompilerParams(use_tc_tiling_on_sc=False),
  )
  h3 = head.reshape(NW, NCH, CH)
  r3 = rel.reshape(NW, NCH, CH)
  t3 = tail.reshape(NW, NCH, CH)
  oh, orr, ot = run(h3, r3, t3, entity_table, relation_table)
  return (oh.reshape(BATCH, EMB_DIM),
          orr.reshape(BATCH, EMB_DIM),
          ot.reshape(BATCH, EMB_DIM))


def kernel(head, rel, tail, entity_table, relation_table):
  return _transe_lookup(head, rel, tail, entity_table, relation_table)
